# Initial kernel scaffold; baseline (speedup 1.0000x reference)
import functools

import jax
import jax.numpy as jnp
from jax import lax
from jax.experimental import pallas as pl
from jax.experimental.pallas import tpu as pltpu
from jax.experimental.pallas import tpu_sc as plsc

B = 4096
N_CAT = 8
CAT_VOCAB = 1000
USER_VOCAB = 7176
ROWS_PER_EX = 10
TOT_ROWS = B * ROWS_PER_EX
NC, NS = 2, 16
NW = NC * NS
RPW = TOT_ROWS // NW
CHUNK = 128
NCHUNK = RPW // CHUNK
GATHER_W = 16
EMB_W = 32 + N_CAT * 16
NUM_W = 64
HID = 128
OUT_W = 64

_sc_mesh = plsc.VectorSubcoreMesh(core_axis_name="c", subcore_axis_name="s")


@functools.partial(
    pl.kernel,
    mesh=_sc_mesh,
    out_type=jax.ShapeDtypeStruct((TOT_ROWS, GATHER_W), jnp.float32),
    scratch_types=[
        pltpu.VMEM((NCHUNK, CHUNK), jnp.int32),
        pltpu.VMEM((RPW, GATHER_W), jnp.float32),
        pltpu.SemaphoreType.DMA,
    ],
)
def _sc_gather(table_hbm, idx_hbm, out_hbm, idx_v, rows_v, sem):
    wid = lax.axis_index("s") * NC + lax.axis_index("c")
    base = wid * RPW
    pltpu.sync_copy(idx_hbm.at[wid], idx_v)
    copies = []
    for j in range(NCHUNK):
        copies.append(
            pltpu.async_copy(
                table_hbm.at[idx_v.at[j]],
                rows_v.at[pl.ds(j * CHUNK, CHUNK)],
                sem,
            )
        )
    for c in copies:
        c.wait()
    pltpu.sync_copy(rows_v, out_hbm.at[pl.ds(base, RPW)])


def _mlp_body(g_ref, n_ref, w1g_ref, w1n_ref, b1_ref, w2_ref, b2_ref, o_ref):
    h = jnp.dot(g_ref[...], w1g_ref[...], preferred_element_type=jnp.float32)
    h = h + jnp.dot(n_ref[...], w1n_ref[...], preferred_element_type=jnp.float32)
    h = jnp.maximum(h + b1_ref[...], 0.0)
    out = jnp.dot(h, w2_ref[...], preferred_element_type=jnp.float32) + b2_ref[...]
    ss = jnp.sum(out * out, axis=1, keepdims=True)
    o_ref[...] = out / jnp.maximum(jnp.sqrt(ss), 1e-12)


def _mlp(g, numeric, w1g, w1n, b1, w2, b2, blk=512):
    grid = (B // blk,)
    return pl.pallas_call(
        _mlp_body,
        grid=grid,
        in_specs=[
            pl.BlockSpec((blk, EMB_W), lambda i: (i, 0)),
            pl.BlockSpec((blk, NUM_W), lambda i: (i, 0)),
            pl.BlockSpec((EMB_W, HID), lambda i: (0, 0)),
            pl.BlockSpec((NUM_W, HID), lambda i: (0, 0)),
            pl.BlockSpec((HID,), lambda i: (0,)),
            pl.BlockSpec((HID, OUT_W), lambda i: (0, 0)),
            pl.BlockSpec((OUT_W,), lambda i: (0,)),
        ],
        out_specs=pl.BlockSpec((blk, OUT_W), lambda i: (i, 0)),
        out_shape=jax.ShapeDtypeStruct((B, OUT_W), jnp.float32),
    )(g, numeric, w1g, w1n, b1, w2, b2)


def kernel(user_ids, user_cat_feats, user_numeric_feats, user_id_table,
           cat_tables, W1, b1, W2, b2):
    uid = user_ids.astype(jnp.int32)
    cf = user_cat_feats.astype(jnp.int32)
    table = jnp.concatenate(
        [user_id_table.reshape(-1, GATHER_W), cat_tables.reshape(-1, GATHER_W)],
        axis=0,
    )
    id_base = USER_VOCAB * 2
    offs = id_base + CAT_VOCAB * jnp.arange(N_CAT, dtype=jnp.int32)
    idx = jnp.concatenate(
        [jnp.stack([2 * uid, 2 * uid + 1], axis=1), cf + offs[None, :]],
        axis=1,
    )
    idx3 = idx.reshape(NW, NCHUNK, CHUNK)
    rows = _sc_gather(table, idx3)
    g = rows.reshape(B, EMB_W)
    return _mlp(g, user_numeric_feats, W1[:EMB_W], W1[EMB_W:], b1, W2, b2)


# speedup vs baseline: 1.8319x; 1.8319x over previous
import functools

import jax
import jax.numpy as jnp
from jax import lax
from jax.experimental import pallas as pl
from jax.experimental.pallas import tpu as pltpu
from jax.experimental.pallas import tpu_sc as plsc

B = 4096
N_CAT = 8
CAT_VOCAB = 1000
USER_VOCAB = 7176
ROWS_PER_EX = 10
TOT_ROWS = B * ROWS_PER_EX
NC, NS = 2, 16
NW = NC * NS
RPW = TOT_ROWS // NW
CHUNK = 128
NCHUNK = RPW // CHUNK
GATHER_W = 16
EMB_W = 32 + N_CAT * 16
NUM_W = 64
HID = 128
OUT_W = 64

@functools.lru_cache(maxsize=None)
def _make_sc_gather():
    mesh = plsc.VectorSubcoreMesh(core_axis_name="c", subcore_axis_name="s")

    @functools.partial(
        pl.kernel,
        mesh=mesh,
        out_type=jax.ShapeDtypeStruct((TOT_ROWS, GATHER_W), jnp.float32),
        scratch_types=[
            pltpu.VMEM((NCHUNK, CHUNK), jnp.int32),
            pltpu.VMEM((RPW, GATHER_W), jnp.float32),
            pltpu.SemaphoreType.DMA,
        ],
        compiler_params=pltpu.CompilerParams(use_tc_tiling_on_sc=False),
    )
    def _sc_gather(table_hbm, idx_hbm, out_hbm, idx_v, rows_v, sem):
        wid = lax.axis_index("s") * NC + lax.axis_index("c")
        base = wid * RPW
        pltpu.sync_copy(idx_hbm.at[wid], idx_v)
        copies = []
        for j in range(NCHUNK):
            copies.append(
                pltpu.async_copy(
                    table_hbm.at[idx_v.at[j]],
                    rows_v.at[pl.ds(j * CHUNK, CHUNK)],
                    sem,
                )
            )
        for c in copies:
            c.wait()
        pltpu.sync_copy(rows_v, out_hbm.at[pl.ds(base, RPW)])

    return _sc_gather


def _mlp_body(g_ref, n_ref, w1g_ref, w1n_ref, b1_ref, w2_ref, b2_ref, o_ref):
    h = jnp.dot(g_ref[...], w1g_ref[...], preferred_element_type=jnp.float32)
    h = h + jnp.dot(n_ref[...], w1n_ref[...], preferred_element_type=jnp.float32)
    h = jnp.maximum(h + b1_ref[...], 0.0)
    out = jnp.dot(h, w2_ref[...], preferred_element_type=jnp.float32) + b2_ref[...]
    ss = jnp.sum(out * out, axis=1, keepdims=True)
    o_ref[...] = out / jnp.maximum(jnp.sqrt(ss), 1e-12)


def _mlp(g, numeric, w1g, w1n, b1, w2, b2, blk=512):
    grid = (B // blk,)
    return pl.pallas_call(
        _mlp_body,
        grid=grid,
        in_specs=[
            pl.BlockSpec((blk, EMB_W), lambda i: (i, 0)),
            pl.BlockSpec((blk, NUM_W), lambda i: (i, 0)),
            pl.BlockSpec((EMB_W, HID), lambda i: (0, 0)),
            pl.BlockSpec((NUM_W, HID), lambda i: (0, 0)),
            pl.BlockSpec((HID,), lambda i: (0,)),
            pl.BlockSpec((HID, OUT_W), lambda i: (0, 0)),
            pl.BlockSpec((OUT_W,), lambda i: (0,)),
        ],
        out_specs=pl.BlockSpec((blk, OUT_W), lambda i: (i, 0)),
        out_shape=jax.ShapeDtypeStruct((B, OUT_W), jnp.float32),
    )(g, numeric, w1g, w1n, b1, w2, b2)


def kernel(user_ids, user_cat_feats, user_numeric_feats, user_id_table,
           cat_tables, W1, b1, W2, b2):
    uid = user_ids.astype(jnp.int32)
    cf = user_cat_feats.astype(jnp.int32)
    table = jnp.concatenate(
        [user_id_table.reshape(-1, GATHER_W), cat_tables.reshape(-1, GATHER_W)],
        axis=0,
    )
    id_base = USER_VOCAB * 2
    offs = id_base + CAT_VOCAB * jnp.arange(N_CAT, dtype=jnp.int32)
    idx = jnp.concatenate(
        [jnp.stack([2 * uid, 2 * uid + 1], axis=1), cf + offs[None, :]],
        axis=1,
    )
    idx3 = idx.reshape(NW, NCHUNK, CHUNK)
    rows = _make_sc_gather()(table, idx3)
    g = rows.reshape(B, EMB_W)
    return _mlp(g, user_numeric_feats, W1[:EMB_W], W1[EMB_W:], b1, W2, b2)


# speedup vs baseline: 2.9389x; 1.6043x over previous
import functools

import jax
import jax.numpy as jnp
from jax import lax
from jax.experimental import pallas as pl
from jax.experimental.pallas import tpu as pltpu
from jax.experimental.pallas import tpu_sc as plsc

B = 4096
N_CAT = 8
NC, NS = 2, 16
NW = NC * NS
BPW = B // NW
ID_W = 32
NUM_W = 64
CAT_W = 16
EMB_W = ID_W + N_CAT * CAT_W
HID = 128
OUT_W = 64


@functools.lru_cache(maxsize=None)
def _make_sc_gather():
    mesh = plsc.VectorSubcoreMesh(core_axis_name="c", subcore_axis_name="s")

    @functools.partial(
        pl.kernel,
        mesh=mesh,
        out_type=(
            jax.ShapeDtypeStruct((B, 128), jnp.float32),
            jax.ShapeDtypeStruct((B, 128), jnp.float32),
        ),
        scratch_types=[
            pltpu.VMEM((BPW,), jnp.int32),
            pltpu.VMEM((N_CAT, BPW), jnp.int32),
            pltpu.VMEM((BPW, ID_W), jnp.float32),
            pltpu.VMEM((BPW, NUM_W), jnp.float32),
            pltpu.VMEM((N_CAT, BPW, CAT_W), jnp.float32),
            pltpu.SemaphoreType.DMA,
        ],
        compiler_params=pltpu.CompilerParams(use_tc_tiling_on_sc=False),
    )
    def _sc(idt_hbm, cat_hbm, uid_hbm, cft_hbm, num_hbm, outc_hbm, outn_hbm,
            ids_v, cidx_v, id_rows, num_v, cat_rows, sem):
        wid = lax.axis_index("s") * NC + lax.axis_index("c")
        base = wid * BPW
        pltpu.sync_copy(uid_hbm.at[pl.ds(base, BPW)], ids_v)
        pltpu.sync_copy(cft_hbm.at[:, pl.ds(base, BPW)], cidx_v)
        copies = [
            pltpu.async_copy(num_hbm.at[pl.ds(base, BPW)], num_v, sem),
            pltpu.async_copy(idt_hbm.at[ids_v], id_rows, sem),
        ]
        for s in range(N_CAT):
            copies.append(
                pltpu.async_copy(cat_hbm.at[s].at[cidx_v.at[s]],
                                 cat_rows.at[s], sem)
            )
        for c in copies:
            c.wait()
        pltpu.sync_copy(id_rows, outn_hbm.at[pl.ds(base, BPW), pl.ds(0, ID_W)])
        pltpu.sync_copy(num_v,
                        outn_hbm.at[pl.ds(base, BPW), pl.ds(ID_W, NUM_W)])
        for s in range(N_CAT):
            pltpu.sync_copy(
                cat_rows.at[s],
                outc_hbm.at[pl.ds(base, BPW), pl.ds(CAT_W * s, CAT_W)],
            )

    return _sc


def _mlp_body(c_ref, n_ref, w1_ref, b1_ref, w2_ref, b2_ref, o_ref):
    c = c_ref[...]
    n = n_ref[...]
    w1 = w1_ref[...]
    h = jnp.dot(n[:, 0:ID_W], w1[0:ID_W], preferred_element_type=jnp.float32)
    h = h + jnp.dot(c, w1[ID_W:ID_W + N_CAT * CAT_W],
                    preferred_element_type=jnp.float32)
    h = h + jnp.dot(n[:, ID_W:ID_W + NUM_W], w1[EMB_W:],
                    preferred_element_type=jnp.float32)
    h = jnp.maximum(h + b1_ref[...], 0.0)
    out = jnp.dot(h, w2_ref[...], preferred_element_type=jnp.float32) + b2_ref[...]
    ss = jnp.sum(out * out, axis=1, keepdims=True)
    o_ref[...] = out / jnp.maximum(jnp.sqrt(ss), 1e-12)


def _mlp(outc, outn, w1, b1, w2, b2, blk=512):
    grid = (B // blk,)
    return pl.pallas_call(
        _mlp_body,
        grid=grid,
        in_specs=[
            pl.BlockSpec((blk, 128), lambda i: (i, 0)),
            pl.BlockSpec((blk, 128), lambda i: (i, 0)),
            pl.BlockSpec((EMB_W + NUM_W, HID), lambda i: (0, 0)),
            pl.BlockSpec((HID,), lambda i: (0,)),
            pl.BlockSpec((HID, OUT_W), lambda i: (0, 0)),
            pl.BlockSpec((OUT_W,), lambda i: (0,)),
        ],
        out_specs=pl.BlockSpec((blk, OUT_W), lambda i: (i, 0)),
        out_shape=jax.ShapeDtypeStruct((B, OUT_W), jnp.float32),
    )(outc, outn, w1, b1, w2, b2)


def kernel(user_ids, user_cat_feats, user_numeric_feats, user_id_table,
           cat_tables, W1, b1, W2, b2):
    uid = user_ids.astype(jnp.int32)
    cft = user_cat_feats.astype(jnp.int32).T
    outc, outn = _make_sc_gather()(
        user_id_table, cat_tables, uid, cft, user_numeric_feats)
    return _mlp(outc, outn, W1, b1, W2, b2)


# speedup vs baseline: 3.2182x; 1.0951x over previous
import functools

import jax
import jax.numpy as jnp
from jax import lax
from jax.experimental import pallas as pl
from jax.experimental.pallas import tpu as pltpu
from jax.experimental.pallas import tpu_sc as plsc

B = 4096
N_CAT = 8
NC, NS = 2, 16
NW = NC * NS
BPW = B // NW
ID_W = 32
NUM_W = 64
CAT_W = 16
EMB_W = ID_W + N_CAT * CAT_W
HID = 128
OUT_W = 64


@functools.lru_cache(maxsize=None)
def _make_sc_gather():
    mesh = plsc.VectorSubcoreMesh(core_axis_name="c", subcore_axis_name="s")

    @functools.partial(
        pl.kernel,
        mesh=mesh,
        out_type=(
            jax.ShapeDtypeStruct((B, 128), jnp.float32),
            jax.ShapeDtypeStruct((B, 128), jnp.float32),
        ),
        scratch_types=[
            pltpu.VMEM((BPW,), jnp.int32),
            pltpu.VMEM((N_CAT, BPW), jnp.int32),
            pltpu.VMEM((BPW, ID_W), jnp.float32),
            pltpu.VMEM((BPW, NUM_W), jnp.float32),
            pltpu.VMEM((N_CAT, BPW, CAT_W), jnp.float32),
            pltpu.SemaphoreType.DMA,
            pltpu.SemaphoreType.DMA,
        ],
        compiler_params=pltpu.CompilerParams(use_tc_tiling_on_sc=False),
    )
    def _sc(idt_hbm, cat_hbm, uid_hbm, cft_hbm, num_hbm, outc_hbm, outn_hbm,
            ids_v, cidx_v, id_rows, num_v, cat_rows, sem, sem2):
        wid = lax.axis_index("s") * NC + lax.axis_index("c")
        base = wid * BPW
        idx_loads = [
            pltpu.async_copy(uid_hbm.at[pl.ds(base, BPW)], ids_v, sem),
            pltpu.async_copy(cft_hbm.at[:, pl.ds(base, BPW)], cidx_v, sem),
        ]
        num_load = pltpu.async_copy(num_hbm.at[pl.ds(base, BPW)], num_v, sem2)
        for c in idx_loads:
            c.wait()
        copies = [pltpu.async_copy(idt_hbm.at[ids_v], id_rows, sem)]
        for s in range(N_CAT):
            copies.append(
                pltpu.async_copy(cat_hbm.at[s].at[cidx_v.at[s]],
                                 cat_rows.at[s], sem)
            )
        copies[0].wait()
        stores = [
            pltpu.async_copy(id_rows,
                             outn_hbm.at[pl.ds(base, BPW), pl.ds(0, ID_W)],
                             sem2)
        ]
        num_load.wait()
        stores.append(
            pltpu.async_copy(num_v,
                             outn_hbm.at[pl.ds(base, BPW), pl.ds(ID_W, NUM_W)],
                             sem2)
        )
        for s in range(N_CAT):
            copies[1 + s].wait()
            stores.append(
                pltpu.async_copy(
                    cat_rows.at[s],
                    outc_hbm.at[pl.ds(base, BPW), pl.ds(CAT_W * s, CAT_W)],
                    sem2,
                )
            )
        for st in stores:
            st.wait()

    return _sc


def _mlp_body(c_ref, n_ref, w1_ref, b1_ref, w2_ref, b2_ref, o_ref):
    c = c_ref[...]
    n = n_ref[...]
    w1 = w1_ref[...]
    h = jnp.dot(n[:, 0:ID_W], w1[0:ID_W], preferred_element_type=jnp.float32)
    h = h + jnp.dot(c, w1[ID_W:ID_W + N_CAT * CAT_W],
                    preferred_element_type=jnp.float32)
    h = h + jnp.dot(n[:, ID_W:ID_W + NUM_W], w1[EMB_W:],
                    preferred_element_type=jnp.float32)
    h = jnp.maximum(h + b1_ref[...], 0.0)
    out = jnp.dot(h, w2_ref[...], preferred_element_type=jnp.float32) + b2_ref[...]
    ss = jnp.sum(out * out, axis=1, keepdims=True)
    o_ref[...] = out / jnp.maximum(jnp.sqrt(ss), 1e-12)


def _mlp(outc, outn, w1, b1, w2, b2, blk=1024):
    grid = (B // blk,)
    return pl.pallas_call(
        _mlp_body,
        grid=grid,
        in_specs=[
            pl.BlockSpec((blk, 128), lambda i: (i, 0)),
            pl.BlockSpec((blk, 128), lambda i: (i, 0)),
            pl.BlockSpec((EMB_W + NUM_W, HID), lambda i: (0, 0)),
            pl.BlockSpec((HID,), lambda i: (0,)),
            pl.BlockSpec((HID, OUT_W), lambda i: (0, 0)),
            pl.BlockSpec((OUT_W,), lambda i: (0,)),
        ],
        out_specs=pl.BlockSpec((blk, OUT_W), lambda i: (i, 0)),
        out_shape=jax.ShapeDtypeStruct((B, OUT_W), jnp.float32),
    )(outc, outn, w1, b1, w2, b2)


def kernel(user_ids, user_cat_feats, user_numeric_feats, user_id_table,
           cat_tables, W1, b1, W2, b2):
    uid = user_ids.astype(jnp.int32)
    cft = user_cat_feats.astype(jnp.int32).T
    outc, outn = _make_sc_gather()(
        user_id_table, cat_tables, uid, cft, user_numeric_feats)
    return _mlp(outc, outn, W1, b1, W2, b2)


# speedup vs baseline: 3.3085x; 1.0280x over previous
import functools

import jax
import jax.numpy as jnp
from jax import lax
from jax.experimental import pallas as pl
from jax.experimental.pallas import tpu as pltpu
from jax.experimental.pallas import tpu_sc as plsc

B = 4096
N_CAT = 8
NC, NS = 2, 16
NW = NC * NS
BPW = B // NW
ID_W = 32
NUM_W = 64
CAT_W = 16
EMB_W = ID_W + N_CAT * CAT_W
HID = 128
OUT_W = 64


@functools.lru_cache(maxsize=None)
def _make_sc_gather():
    mesh = plsc.VectorSubcoreMesh(core_axis_name="c", subcore_axis_name="s")

    @functools.partial(
        pl.kernel,
        mesh=mesh,
        out_type=(
            jax.ShapeDtypeStruct((B, 128), jnp.float32),
            jax.ShapeDtypeStruct((B, 128), jnp.float32),
        ),
        scratch_types=[
            pltpu.VMEM((BPW,), jnp.int32),
            pltpu.VMEM((N_CAT, BPW), jnp.int32),
            pltpu.VMEM((BPW, ID_W), jnp.float32),
            pltpu.VMEM((BPW, NUM_W), jnp.float32),
            pltpu.VMEM((N_CAT, BPW, CAT_W), jnp.float32),
            pltpu.SemaphoreType.DMA,
            pltpu.SemaphoreType.DMA,
        ],
        compiler_params=pltpu.CompilerParams(use_tc_tiling_on_sc=False),
    )
    def _sc(idt_hbm, cat_hbm, uid_hbm, cft_hbm, num_hbm, outc_hbm, outn_hbm,
            ids_v, cidx_v, id_rows, num_v, cat_rows, sem, sem2):
        wid = lax.axis_index("s") * NC + lax.axis_index("c")
        base = wid * BPW
        idx_loads = [
            pltpu.async_copy(uid_hbm.at[pl.ds(base, BPW)], ids_v, sem),
            pltpu.async_copy(cft_hbm.at[:, pl.ds(base, BPW)], cidx_v, sem),
        ]
        num_load = pltpu.async_copy(num_hbm.at[pl.ds(base, BPW)], num_v, sem2)
        for c in idx_loads:
            c.wait()
        copies = [pltpu.async_copy(idt_hbm.at[ids_v], id_rows, sem)]
        for s in range(N_CAT):
            copies.append(
                pltpu.async_copy(cat_hbm.at[s].at[cidx_v.at[s]],
                                 cat_rows.at[s], sem)
            )
        copies[0].wait()
        stores = [
            pltpu.async_copy(id_rows,
                             outn_hbm.at[pl.ds(base, BPW), pl.ds(0, ID_W)],
                             sem2)
        ]
        num_load.wait()
        stores.append(
            pltpu.async_copy(num_v,
                             outn_hbm.at[pl.ds(base, BPW), pl.ds(ID_W, NUM_W)],
                             sem2)
        )
        for s in range(N_CAT):
            copies[1 + s].wait()
            stores.append(
                pltpu.async_copy(
                    cat_rows.at[s],
                    outc_hbm.at[pl.ds(base, BPW), pl.ds(CAT_W * s, CAT_W)],
                    sem2,
                )
            )
        for st in stores:
            st.wait()

    return _sc


def _mlp_body(c_ref, n_ref, w1_ref, b1_ref, w2_ref, b2_ref, o_ref):
    c = c_ref[...]
    n = n_ref[...]
    w1 = w1_ref[...]
    h = jnp.dot(n[:, 0:ID_W], w1[0:ID_W], preferred_element_type=jnp.float32)
    h = h + jnp.dot(c, w1[ID_W:ID_W + N_CAT * CAT_W],
                    preferred_element_type=jnp.float32)
    h = h + jnp.dot(n[:, ID_W:ID_W + NUM_W], w1[EMB_W:],
                    preferred_element_type=jnp.float32)
    h = jnp.maximum(h + b1_ref[...], 0.0)
    out = jnp.dot(h, w2_ref[...], preferred_element_type=jnp.float32) + b2_ref[...]
    ss = jnp.sum(out * out, axis=1, keepdims=True)
    o_ref[...] = out / jnp.maximum(jnp.sqrt(ss), 1e-12)


def _mlp(outc, outn, w1, b1, w2, b2, blk=2048):
    grid = (B // blk,)
    return pl.pallas_call(
        _mlp_body,
        grid=grid,
        in_specs=[
            pl.BlockSpec((blk, 128), lambda i: (i, 0)),
            pl.BlockSpec((blk, 128), lambda i: (i, 0)),
            pl.BlockSpec((EMB_W + NUM_W, HID), lambda i: (0, 0)),
            pl.BlockSpec((HID,), lambda i: (0,)),
            pl.BlockSpec((HID, OUT_W), lambda i: (0, 0)),
            pl.BlockSpec((OUT_W,), lambda i: (0,)),
        ],
        out_specs=pl.BlockSpec((blk, OUT_W), lambda i: (i, 0)),
        out_shape=jax.ShapeDtypeStruct((B, OUT_W), jnp.float32),
    )(outc, outn, w1, b1, w2, b2)


def kernel(user_ids, user_cat_feats, user_numeric_feats, user_id_table,
           cat_tables, W1, b1, W2, b2):
    uid = user_ids.astype(jnp.int32)
    cft = user_cat_feats.astype(jnp.int32).T
    outc, outn = _make_sc_gather()(
        user_id_table, cat_tables, uid, cft, user_numeric_feats)
    return _mlp(outc, outn, W1, b1, W2, b2)
